# baseline (speedup 1.0000x reference)
import functools

import jax
import jax.numpy as jnp
from jax.experimental import pallas as pl
from jax.experimental.pallas import tpu as pltpu

_N_GROUP = 8
_GS = 8
_TOPK_GROUP = 4
_TOP_K = 8
_SCALE = 2.5
_E = 64
_NBUF = 6


def _rot_rows_within_group(v, s):
    r = jax.lax.broadcasted_iota(jnp.int32, v.shape, 0) % _GS
    w = jnp.roll(v, -s, axis=0)
    u = jnp.roll(v, _GS - s, axis=0)
    return jnp.where(r < _GS - s, w, u)


def _compute(h, w, bias, o_ref):
    logits_t = jax.lax.dot_general(
        w, h, (((1,), (1,)), ((), ())), preferred_element_type=jnp.float32)
    scores = jax.nn.sigmoid(logits_t)
    swb = scores + bias.reshape(_E, 1)
    E, bT = swb.shape
    row = jax.lax.broadcasted_iota(jnp.int32, (E, 1), 0)

    m1 = swb
    m2 = jnp.full_like(swb, -jnp.inf)
    for s in (1, 2, 4):
        o1 = _rot_rows_within_group(m1, s)
        o2 = _rot_rows_within_group(m2, s)
        nm1 = jnp.maximum(m1, o1)
        nm2 = jnp.maximum(jnp.minimum(m1, o1), jnp.maximum(m2, o2))
        m1, m2 = nm1, nm2
    gs_full = m1 + m2

    gs8 = gs_full.reshape(_N_GROUP, _GS, bT)[:, 0, :]
    g8 = jax.lax.broadcasted_iota(jnp.int32, (_N_GROUP, 1), 0)
    grank = jnp.zeros_like(gs8)
    for k in range(1, _N_GROUP):
        other = jnp.roll(gs8, -k, axis=0)
        tiebreak = ((g8 + k) % _N_GROUP) < g8
        beats = (other > gs8) | ((other == gs8) & tiebreak)
        grank = grank + jnp.where(beats, 1.0, 0.0)
    gsel = jnp.broadcast_to((grank < _TOPK_GROUP)[:, None, :],
                            (_N_GROUP, _GS, bT)).reshape(E, bT)
    masked = jnp.where(gsel, swb, 0.0)

    remaining = masked
    selmask = jnp.zeros_like(swb, dtype=jnp.bool_)
    for it in range(_TOP_K):
        m = jnp.max(remaining, axis=0, keepdims=True)
        cand = remaining == m
        idx = jnp.min(jnp.where(cand, row, E), axis=0, keepdims=True)
        pick = row == idx
        selmask = selmask | pick
        if it < _TOP_K - 1:
            remaining = jnp.where(pick, -jnp.inf, remaining)

    selected = jnp.where(selmask, scores, 0.0)
    ssum = jnp.sum(selected, axis=0, keepdims=True) + 1e-20
    out = selected / ssum * _SCALE
    o_ref[...] = out.T


def _gate_kernel(n, bT, h_hbm, w_ref, b_ref, o_ref,
                 buf0, buf1, buf2, buf3, buf4, buf5,
                 sem0, sem1, sem2, sem3, sem4, sem5):
    i = pl.program_id(0)
    bufs = (buf0, buf1, buf2, buf3, buf4, buf5)
    sems = (sem0, sem1, sem2, sem3, sem4, sem5)

    @pl.when(i == 0)
    def _():
        for q in range(_NBUF):
            if q < n:
                pltpu.make_async_copy(
                    h_hbm.at[pl.ds(q * bT, bT), :], bufs[q], sems[q]).start()

    p = jax.lax.rem(i, _NBUF)
    for q in range(_NBUF):
        @pl.when(p == q)
        def _(q=q):
            pltpu.make_async_copy(
                h_hbm.at[pl.ds(i * bT, bT), :], bufs[q], sems[q]).wait()
            _compute(bufs[q][...], w_ref[...], b_ref[...], o_ref)

            @pl.when(i + _NBUF < n)
            def _():
                pltpu.make_async_copy(
                    h_hbm.at[pl.ds((i + _NBUF) * bT, bT), :],
                    bufs[q], sems[q]).start()


def kernel(hidden_states, weight, e_score_correction_bias):
    T, H = hidden_states.shape
    E = weight.shape[0]
    bT = min(512, T)
    n = T // bT
    bias2 = e_score_correction_bias.reshape(1, E).astype(jnp.float32)
    return pl.pallas_call(
        functools.partial(_gate_kernel, n, bT),
        grid=(n,),
        in_specs=[
            pl.BlockSpec(memory_space=pl.ANY),
            pl.BlockSpec((E, H), lambda i: (0, 0)),
            pl.BlockSpec((1, E), lambda i: (0, 0)),
        ],
        out_specs=pl.BlockSpec((bT, E), lambda i: (i, 0)),
        out_shape=jax.ShapeDtypeStruct((T, E), jnp.float32),
        scratch_shapes=[pltpu.VMEM((bT, H), jnp.float32)] * _NBUF
                       + [pltpu.SemaphoreType.DMA] * _NBUF,
        compiler_params=pltpu.CompilerParams(
            dimension_semantics=("arbitrary",)),
    )(hidden_states.astype(jnp.float32), weight.astype(jnp.float32), bias2)


# speedup vs baseline: 1.0433x; 1.0433x over previous
import jax
import jax.numpy as jnp
from jax.experimental import pallas as pl
from jax.experimental.pallas import tpu as pltpu

_N_GROUP = 8
_GS = 8
_TOPK_GROUP = 4
_TOP_K = 8
_SCALE = 2.5
_E = 64


def _rot_rows_within_group(v, s):
    r = jax.lax.broadcasted_iota(jnp.int32, v.shape, 0) % _GS
    w = jnp.roll(v, -s, axis=0)
    u = jnp.roll(v, _GS - s, axis=0)
    return jnp.where(r < _GS - s, w, u)


def _gate_kernel(h_ref, w_ref, b_ref, o_ref):
    h = h_ref[...]
    w = w_ref[...]
    logits_t = jax.lax.dot_general(
        w, h, (((1,), (1,)), ((), ())), preferred_element_type=jnp.float32)
    scores = jax.nn.sigmoid(logits_t)
    swb = scores + b_ref[...].reshape(_E, 1)
    E, bT = swb.shape
    row = jax.lax.broadcasted_iota(jnp.int32, (E, 1), 0)
    g = row // _GS

    m1 = swb
    m2 = jnp.full_like(swb, -jnp.inf)
    for s in (1, 2, 4):
        o1 = _rot_rows_within_group(m1, s)
        o2 = _rot_rows_within_group(m2, s)
        nm1 = jnp.maximum(m1, o1)
        nm2 = jnp.maximum(jnp.minimum(m1, o1), jnp.maximum(m2, o2))
        m1, m2 = nm1, nm2
    gs_full = m1 + m2

    gs8 = gs_full.reshape(_N_GROUP, _GS, bT)[:, 0, :]
    g8 = jax.lax.broadcasted_iota(jnp.int32, (_N_GROUP, 1), 0)
    grank = jnp.zeros_like(gs8)
    for k in range(1, _N_GROUP):
        other = jnp.roll(gs8, -k, axis=0)
        tiebreak = ((g8 + k) % _N_GROUP) < g8
        beats = (other > gs8) | ((other == gs8) & tiebreak)
        grank = grank + jnp.where(beats, 1.0, 0.0)
    gsel = jnp.broadcast_to((grank < _TOPK_GROUP)[:, None, :],
                            (_N_GROUP, _GS, bT)).reshape(E, bT)
    masked = jnp.where(gsel, swb, 0.0)

    remaining = masked
    selmask = jnp.zeros_like(swb, dtype=jnp.bool_)
    for it in range(_TOP_K):
        m = jnp.max(remaining, axis=0, keepdims=True)
        cand = remaining == m
        idx = jnp.min(jnp.where(cand, row, E), axis=0, keepdims=True)
        pick = row == idx
        selmask = selmask | pick
        if it < _TOP_K - 1:
            remaining = jnp.where(pick, -jnp.inf, remaining)

    selected = jnp.where(selmask, scores, 0.0)
    ssum = jnp.sum(selected, axis=0, keepdims=True) + 1e-20
    out = selected / ssum * _SCALE
    o_ref[...] = out.T


def kernel(hidden_states, weight, e_score_correction_bias):
    T, H = hidden_states.shape
    E = weight.shape[0]
    bT = min(1024, T)
    bias2 = e_score_correction_bias.reshape(1, E).astype(jnp.float32)
    return pl.pallas_call(
        _gate_kernel,
        grid=(T // bT,),
        in_specs=[
            pl.BlockSpec((bT, H), lambda i: (i, 0)),
            pl.BlockSpec((E, H), lambda i: (0, 0)),
            pl.BlockSpec((1, E), lambda i: (0, 0)),
        ],
        out_specs=pl.BlockSpec((bT, E), lambda i: (i, 0)),
        out_shape=jax.ShapeDtypeStruct((T, E), jnp.float32),
        compiler_params=pltpu.CompilerParams(
            dimension_semantics=("parallel",)),
    )(hidden_states.astype(jnp.float32), weight.astype(jnp.float32), bias2)
